# baseline (speedup 1.0000x reference)
import functools

import jax
import jax.numpy as jnp
from jax import lax
from jax.experimental import pallas as pl
from jax.experimental.pallas import tpu as pltpu
from jax.experimental.pallas import tpu_sc as plsc

B = 4096
L = 200
D = 64
SCALE = float(D) ** 0.5
EPS = 1.1754943508222875e-38
N_ELEM = L * D

_NC = 2
_NS = 16
NW = _NC * _NS
BPW = B // NW
NBUF = 4

_mesh = plsc.VectorSubcoreMesh(core_axis_name="c", subcore_axis_name="s")


@functools.partial(
    pl.kernel,
    mesh=_mesh,
    compiler_params=pltpu.CompilerParams(use_tc_tiling_on_sc=False, needs_layout_passes=False),
    out_type=[
        jax.ShapeDtypeStruct((L, NW, D, BPW), jnp.float32),
        jax.ShapeDtypeStruct((NW, BPW), jnp.float32),
        jax.ShapeDtypeStruct((NW, BPW), jnp.float32),
    ],
    scratch_types=[
        pltpu.VMEM((L, BPW), jnp.int32),
        pltpu.VMEM((NBUF, BPW, D), jnp.float32),
        pltpu.VMEM((NBUF, D, BPW), jnp.float32),
        pltpu.VMEM((D, BPW), jnp.float32),
        pltpu.VMEM((D, BPW), jnp.float32),
        pltpu.VMEM((2, BPW), jnp.float32),
        pltpu.SemaphoreType.DMA((NBUF,)),
        pltpu.SemaphoreType.DMA((NBUF,)),
    ],
)
def _sc_gather_stats(idx_hbm, emb_hbm, raw_hbm, s_hbm, q_hbm,
                     idx_v, rows_v, trows_v, acc_s, acc_q, sq_v, gsem, wsem):
    wid = lax.axis_index("s") * _NC + lax.axis_index("c")

    pltpu.sync_copy(idx_hbm.at[wid], idx_v)

    zeros = jnp.zeros((16,), jnp.float32)

    def zero_body(d, _):
        for c in range(BPW // 16):
            acc_s[d, pl.ds(c * 16, 16)] = zeros
            acc_q[d, pl.ds(c * 16, 16)] = zeros
        return 0
    lax.fori_loop(0, D, zero_body, 0)

    def fire_gather(l, j):
        pltpu.async_copy(emb_hbm.at[idx_v.at[l]], rows_v.at[j], gsem.at[j])

    def fire_write(l, j):
        pltpu.async_copy(trows_v.at[j], raw_hbm.at[l, wid], wsem.at[j])

    def wait_gather(j):
        pltpu.make_async_copy(emb_hbm.at[idx_v.at[0]], rows_v.at[j],
                              gsem.at[j]).wait()

    def wait_write(j):
        pltpu.make_async_copy(trows_v.at[j], raw_hbm.at[0, 0],
                              wsem.at[j]).wait()

    lanes = lax.iota(jnp.int32, 16)

    def transpose_acc(j):
        rows2d = rows_v.at[j]

        def d_body(d, _):
            dsp = jnp.full((16,), 0, jnp.int32) + d
            for c in range(BPW // 16):
                lb = lanes + (c * 16)
                x = plsc.load_gather(rows2d, [lb, dsp])
                trows_v[j, d, pl.ds(c * 16, 16)] = x
                acc_s[d, pl.ds(c * 16, 16)] += x
                acc_q[d, pl.ds(c * 16, 16)] += x * x
            return 0
        lax.fori_loop(0, D, d_body, 0)

    def step(l, j, jprev, fire, wait_w):
        if fire:
            fire_gather(l + NBUF - 1, jprev)
        wait_gather(j)
        if wait_w:
            wait_write(j)
        transpose_acc(j)
        fire_write(l, j)

    for j in range(NBUF - 1):
        fire_gather(j, j)

    for l in range(NBUF):
        step(l, l % NBUF, (l - 1) % NBUF, True, False)

    def chunk(c, _):
        base = c * NBUF
        for j in range(NBUF):
            step(base + j, j, (j - 1) % NBUF, True, True)
        return 0
    lax.fori_loop(1, (L // NBUF) - 1, chunk, 0)

    step(196, 0, 3, True, True)
    for l in (197, 198, 199):
        step(l, l % NBUF, (l - 1) % NBUF, False, True)

    for j in range(NBUF):
        wait_write(j)

    def red_init(_c):
        for c in range(BPW // 16):
            sq_v[0, pl.ds(c * 16, 16)] = zeros
            sq_v[1, pl.ds(c * 16, 16)] = zeros
    red_init(0)

    def red_body(d, _):
        for c in range(BPW // 16):
            sq_v[0, pl.ds(c * 16, 16)] += acc_s[d, pl.ds(c * 16, 16)]
            sq_v[1, pl.ds(c * 16, 16)] += acc_q[d, pl.ds(c * 16, 16)]
        return 0
    lax.fori_loop(0, D, red_body, 0)

    pltpu.sync_copy(sq_v.at[0], s_hbm.at[wid])
    pltpu.sync_copy(sq_v.at[1], q_hbm.at[wid])


def _finalize_body(s_ref, q_ref, a_ref, c_ref):
    s = s_ref[:, :]
    q = q_ref[:, :]
    n = jnp.float32(N_ELEM)
    mean = s / n
    var = (q - s * s / n) / (n - 1.0)
    std = jnp.sqrt(var)
    inv = SCALE / (SCALE * std + EPS)
    a_ref[:, :] = inv
    c_ref[:, :] = -mean * inv


_finalize = pl.pallas_call(
    _finalize_body,
    out_shape=[
        jax.ShapeDtypeStruct((NW, BPW), jnp.float32),
        jax.ShapeDtypeStruct((NW, BPW), jnp.float32),
    ],
)

_LBLK = 25


def _norm_body(x_ref, a_ref, c_ref, o_ref):
    av = a_ref[0, 0]
    cv = c_ref[0, 0]
    o_ref[...] = x_ref[:, 0] * av + cv


_norm = pl.pallas_call(
    _norm_body,
    grid=(L // _LBLK, NW),
    in_specs=[
        pl.BlockSpec((_LBLK, 1, D, BPW), lambda i, w: (i, w, 0, 0)),
        pl.BlockSpec((1, 1, BPW), lambda i, w: (w, 0, 0)),
        pl.BlockSpec((1, 1, BPW), lambda i, w: (w, 0, 0)),
    ],
    out_specs=pl.BlockSpec((_LBLK, D, BPW), lambda i, w: (i, 0, w)),
    out_shape=jax.ShapeDtypeStruct((L, D, B), jnp.float32),
)


def kernel(inp, emb):
    idx_w = inp.reshape(NW, BPW, L).transpose(0, 2, 1)
    raw, s, q = _sc_gather_stats(idx_w, emb)
    a, c = _finalize(s, q)
    y = _norm(raw, a.reshape(NW, 1, BPW), c.reshape(NW, 1, BPW))
    return jnp.transpose(y, (0, 2, 1))


# speedup vs baseline: 2.3666x; 2.3666x over previous
import functools

import jax
import jax.numpy as jnp
from jax import lax
from jax.experimental import pallas as pl
from jax.experimental.pallas import tpu as pltpu
from jax.experimental.pallas import tpu_sc as plsc

B = 4096
L = 200
D = 64
SCALE = float(D) ** 0.5
EPS = 1.1754943508222875e-38
N_ELEM = L * D

_NC = 2
_NS = 16
NW = _NC * _NS
BPW = B // NW
NBUF = 4

_PERM = [(m // 2) if m % 2 == 0 else 64 + (m // 2) for m in range(BPW)]
_INV_PERM = [0] * BPW
for _m, _k in enumerate(_PERM):
    _INV_PERM[_k] = _m

_mesh = plsc.VectorSubcoreMesh(core_axis_name="c", subcore_axis_name="s")


@functools.partial(
    pl.kernel,
    mesh=_mesh,
    compiler_params=pltpu.CompilerParams(use_tc_tiling_on_sc=False),
    out_type=[
        jax.ShapeDtypeStruct((L * B, D), jnp.float32),
        jax.ShapeDtypeStruct((B, D), jnp.float32),
        jax.ShapeDtypeStruct((B, D), jnp.float32),
    ],
    scratch_types=[
        pltpu.VMEM((L, BPW), jnp.int32),
        pltpu.VMEM((NBUF, BPW, D), jnp.float32),
        pltpu.VMEM((BPW, D), jnp.float32),
        pltpu.VMEM((BPW, D), jnp.float32),
        pltpu.SemaphoreType.DMA((NBUF,)),
        pltpu.SemaphoreType.DMA((NBUF,)),
    ],
)
def _sc_gather_stats(idx_hbm, emb_hbm, raw_hbm, s_hbm, q_hbm,
                     idx_v, rows_v, acc_s, acc_q, gsem, wsem):
    wid = lax.axis_index("s") * _NC + lax.axis_index("c")
    b0 = wid * BPW

    pltpu.sync_copy(idx_hbm.at[wid], idx_v)

    zeros = jnp.zeros((16,), jnp.float32)

    def zero_body(r, _):
        for c in range(D // 16):
            acc_s[r, pl.ds(c * 16, 16)] = zeros
            acc_q[r, pl.ds(c * 16, 16)] = zeros
        return 0
    lax.fori_loop(0, BPW, zero_body, 0)

    def fire_gather(l, j):
        pltpu.async_copy(emb_hbm.at[idx_v.at[l]], rows_v.at[j], gsem.at[j])

    def fire_write(l, j):
        pltpu.async_copy(rows_v.at[j], raw_hbm.at[pl.ds(l * B + b0, BPW)],
                         wsem.at[j])

    def wait_gather(j):
        pltpu.make_async_copy(emb_hbm.at[idx_v.at[0]], rows_v.at[j],
                              gsem.at[j]).wait()

    def wait_write(j):
        pltpu.make_async_copy(rows_v.at[j], raw_hbm.at[pl.ds(b0, BPW)],
                              wsem.at[j]).wait()

    def accumulate(j):
        def r_body(r, _):
            for c in range(D // 16):
                x = rows_v[j, r, pl.ds(c * 16, 16)]
                acc_s[r, pl.ds(c * 16, 16)] += x
                acc_q[r, pl.ds(c * 16, 16)] += x * x
            return 0
        lax.fori_loop(0, BPW, r_body, 0)

    def step(l, j, jprev, first):
        wait_gather(j)
        fire_write(l, j)
        accumulate(j)
        if not first:
            wait_write(jprev)
        fire_gather(l + NBUF - 1, jprev)

    for j in range(NBUF - 1):
        fire_gather(j, j)

    step(0, 0, NBUF - 1, True)
    for l in range(1, NBUF):
        step(l, l % NBUF, (l - 1) % NBUF, False)

    def chunk(c, _):
        base = c * NBUF
        for j in range(NBUF):
            step(base + j, j, (j - 1) % NBUF, False)
        return 0
    lax.fori_loop(1, (L // NBUF) - 1, chunk, 0)

    l = 196
    wait_gather(l % NBUF)
    fire_write(l, l % NBUF)
    accumulate(l % NBUF)
    wait_write((l - 1) % NBUF)
    fire_gather(199, (l - 1) % NBUF)
    for l in (197, 198, 199):
        wait_gather(l % NBUF)
        fire_write(l, l % NBUF)
        accumulate(l % NBUF)

    for j in range(NBUF):
        wait_write(j)

    pltpu.sync_copy(acc_s, s_hbm.at[pl.ds(b0, BPW)])
    pltpu.sync_copy(acc_q, q_hbm.at[pl.ds(b0, BPW)])


def _finalize_body(s_ref, q_ref, a_ref, c_ref):
    s = s_ref[:, :]
    q = q_ref[:, :]
    sum_b = jnp.sum(s, axis=1, keepdims=True)
    sumsq_b = jnp.sum(q, axis=1, keepdims=True)
    n = jnp.float32(N_ELEM)
    mean = sum_b / n
    var = (sumsq_b - sum_b * sum_b / n) / (n - 1.0)
    std = jnp.sqrt(var)
    inv = SCALE / (SCALE * std + EPS)
    a_ref[:, :] = jnp.broadcast_to(inv, (B, D))
    c_ref[:, :] = jnp.broadcast_to(-mean * inv, (B, D))


_finalize = pl.pallas_call(
    _finalize_body,
    out_shape=[
        jax.ShapeDtypeStruct((B, D), jnp.float32),
        jax.ShapeDtypeStruct((B, D), jnp.float32),
    ],
)

_LBLK = 8
_WGRP = 4


def _norm_body(x_ref, a_ref, c_ref, o_ref):
    for k in range(_WGRP):
        x = x_ref[:, k]
        xa = jnp.swapaxes(x[:, :, :64], -1, -2)
        xb = jnp.swapaxes(x[:, :, 64:], -1, -2)
        oc = jnp.concatenate([xa, xb], axis=-1)
        av = a_ref[k, 0]
        cv = c_ref[k, 0]
        o_ref[:, :, pl.ds(k * 128, 128)] = oc * av + cv


_norm = pl.pallas_call(
    _norm_body,
    grid=(L // _LBLK, NW // _WGRP),
    in_specs=[
        pl.BlockSpec((_LBLK, _WGRP, D, BPW), lambda i, w: (i, w, 0, 0)),
        pl.BlockSpec((_WGRP, 1, BPW), lambda i, w: (w, 0, 0)),
        pl.BlockSpec((_WGRP, 1, BPW), lambda i, w: (w, 0, 0)),
    ],
    out_specs=pl.BlockSpec((_LBLK, D, _WGRP * BPW), lambda i, w: (i, 0, w)),
    out_shape=jax.ShapeDtypeStruct((L, D, B), jnp.float32),
)


def kernel(inp, emb):
    perm = jnp.asarray(_PERM, dtype=jnp.int32)
    inv_perm = jnp.asarray(_INV_PERM, dtype=jnp.int32)
    idx_w = inp.reshape(NW, BPW, L)[:, perm, :].transpose(0, 2, 1)
    raw, s, q = _sc_gather_stats(idx_w, emb)
    a_full, c_full = _finalize(s, q)
    a4 = a_full[:, 0].reshape(NW, BPW)[:, inv_perm].reshape(NW, 1, BPW)
    c4 = c_full[:, 0].reshape(NW, BPW)[:, inv_perm].reshape(NW, 1, BPW)
    y = _norm(raw.reshape(L, NW, D, BPW), a4, c4)
    return jnp.transpose(y, (0, 2, 1))


# speedup vs baseline: 2.4649x; 1.0415x over previous
import functools

import jax
import jax.numpy as jnp
from jax import lax
from jax.experimental import pallas as pl
from jax.experimental.pallas import tpu as pltpu
from jax.experimental.pallas import tpu_sc as plsc

B = 4096
L = 200
D = 64
SCALE = float(D) ** 0.5
EPS = 1.1754943508222875e-38
N_ELEM = L * D

_NC = 2
_NS = 16
NW = _NC * _NS
BPW = B // NW
NBUF = 4

_PERM = [(m // 2) if m % 2 == 0 else 64 + (m // 2) for m in range(BPW)]
_INV_PERM = [0] * BPW
for _m, _k in enumerate(_PERM):
    _INV_PERM[_k] = _m

_mesh = plsc.VectorSubcoreMesh(core_axis_name="c", subcore_axis_name="s")


@functools.partial(
    pl.kernel,
    mesh=_mesh,
    compiler_params=pltpu.CompilerParams(use_tc_tiling_on_sc=False),
    out_type=[
        jax.ShapeDtypeStruct((L * B, D), jnp.float32),
        jax.ShapeDtypeStruct((B, D), jnp.float32),
        jax.ShapeDtypeStruct((B, D), jnp.float32),
    ],
    scratch_types=[
        pltpu.VMEM((L, BPW), jnp.int32),
        pltpu.VMEM((NBUF, BPW, D), jnp.float32),
        pltpu.VMEM((BPW, D), jnp.float32),
        pltpu.VMEM((BPW, D), jnp.float32),
        pltpu.SemaphoreType.DMA((NBUF,)),
        pltpu.SemaphoreType.DMA((NBUF,)),
    ],
)
def _sc_gather_stats(idx_hbm, emb_hbm, raw_hbm, s_hbm, q_hbm,
                     idx_v, rows_v, acc_s, acc_q, gsem, wsem):
    wid = lax.axis_index("s") * _NC + lax.axis_index("c")
    b0 = wid * BPW

    pltpu.sync_copy(idx_hbm.at[wid], idx_v)

    zeros = jnp.zeros((16,), jnp.float32)

    def zero_body(r, _):
        for c in range(D // 16):
            acc_s[r, pl.ds(c * 16, 16)] = zeros
            acc_q[r, pl.ds(c * 16, 16)] = zeros
        return 0
    lax.fori_loop(0, BPW, zero_body, 0)

    def fire_gather(l, j):
        pltpu.async_copy(emb_hbm.at[idx_v.at[l]], rows_v.at[j], gsem.at[j])

    def fire_write(l, j):
        pltpu.async_copy(rows_v.at[j], raw_hbm.at[pl.ds(l * B + b0, BPW)],
                         wsem.at[j])

    def wait_gather(j):
        pltpu.make_async_copy(emb_hbm.at[idx_v.at[0]], rows_v.at[j],
                              gsem.at[j]).wait()

    def wait_write(j):
        pltpu.make_async_copy(rows_v.at[j], raw_hbm.at[pl.ds(b0, BPW)],
                              wsem.at[j]).wait()

    def accumulate(j):
        def r_body(r, _):
            for c in range(D // 16):
                x = rows_v[j, r, pl.ds(c * 16, 16)]
                acc_s[r, pl.ds(c * 16, 16)] += x
                acc_q[r, pl.ds(c * 16, 16)] += x * x
            return 0
        lax.fori_loop(0, BPW, r_body, 0)

    def step(l, j, jprev, first):
        wait_gather(j)
        fire_write(l, j)
        accumulate(j)
        if not first:
            wait_write(jprev)
        fire_gather(l + NBUF - 1, jprev)

    for j in range(NBUF - 1):
        fire_gather(j, j)

    step(0, 0, NBUF - 1, True)
    for l in range(1, NBUF):
        step(l, l % NBUF, (l - 1) % NBUF, False)

    def chunk(c, _):
        base = c * NBUF
        for j in range(NBUF):
            step(base + j, j, (j - 1) % NBUF, False)
        return 0
    lax.fori_loop(1, (L // NBUF) - 1, chunk, 0)

    l = 196
    wait_gather(l % NBUF)
    fire_write(l, l % NBUF)
    accumulate(l % NBUF)
    wait_write((l - 1) % NBUF)
    fire_gather(199, (l - 1) % NBUF)
    for l in (197, 198, 199):
        wait_gather(l % NBUF)
        fire_write(l, l % NBUF)
        accumulate(l % NBUF)

    for j in range(NBUF):
        wait_write(j)

    pltpu.sync_copy(acc_s, s_hbm.at[pl.ds(b0, BPW)])
    pltpu.sync_copy(acc_q, q_hbm.at[pl.ds(b0, BPW)])


def _finalize_body(s_ref, q_ref, a_ref, c_ref):
    s = s_ref[...]
    q = q_ref[...]
    se = jnp.sum(s[:, :, :D], axis=2)
    so = jnp.sum(s[:, :, D:], axis=2)
    qe = jnp.sum(q[:, :, :D], axis=2)
    qo = jnp.sum(q[:, :, D:], axis=2)
    sum_b = jnp.concatenate([se, so], axis=-1)
    sumsq_b = jnp.concatenate([qe, qo], axis=-1)
    n = jnp.float32(N_ELEM)
    mean = sum_b / n
    var = (sumsq_b - sum_b * sum_b / n) / (n - 1.0)
    std = jnp.sqrt(var)
    inv = SCALE / (SCALE * std + EPS)
    a_ref[:, 0, :] = inv
    c_ref[:, 0, :] = -mean * inv


_finalize = pl.pallas_call(
    _finalize_body,
    out_shape=[
        jax.ShapeDtypeStruct((NW, 1, BPW), jnp.float32),
        jax.ShapeDtypeStruct((NW, 1, BPW), jnp.float32),
    ],
)

_LBLK = 8
_WGRP = 8


def _norm_body(x_ref, a_ref, c_ref, o_ref):
    for k in range(_WGRP):
        x = x_ref[:, k]
        xa = jnp.swapaxes(x[:, :, :64], -1, -2)
        xb = jnp.swapaxes(x[:, :, 64:], -1, -2)
        oc = jnp.concatenate([xa, xb], axis=-1)
        av = a_ref[k, 0]
        cv = c_ref[k, 0]
        o_ref[:, :, pl.ds(k * 128, 128)] = oc * av + cv


_norm = pl.pallas_call(
    _norm_body,
    grid=(L // _LBLK, NW // _WGRP),
    in_specs=[
        pl.BlockSpec((_LBLK, _WGRP, D, BPW), lambda i, w: (i, w, 0, 0)),
        pl.BlockSpec((_WGRP, 1, BPW), lambda i, w: (w, 0, 0)),
        pl.BlockSpec((_WGRP, 1, BPW), lambda i, w: (w, 0, 0)),
    ],
    out_specs=pl.BlockSpec((_LBLK, D, _WGRP * BPW), lambda i, w: (i, 0, w)),
    out_shape=jax.ShapeDtypeStruct((L, D, B), jnp.float32),
)


def kernel(inp, emb):
    perm = jnp.asarray(_PERM, dtype=jnp.int32)
    idx_w = inp.reshape(NW, BPW, L)[:, perm, :].transpose(0, 2, 1)
    raw, s, q = _sc_gather_stats(idx_w, emb)
    a4, c4 = _finalize(s.reshape(NW, D, BPW), q.reshape(NW, D, BPW))
    y = _norm(raw.reshape(L, NW, D, BPW), a4, c4)
    return jnp.transpose(y, (0, 2, 1))


# speedup vs baseline: 2.6897x; 1.0912x over previous
import functools

import jax
import jax.numpy as jnp
from jax import lax
from jax.experimental import pallas as pl
from jax.experimental.pallas import tpu as pltpu
from jax.experimental.pallas import tpu_sc as plsc

B = 4096
L = 200
D = 64
SCALE = float(D) ** 0.5
EPS = 1.1754943508222875e-38
N_ELEM = L * D

_NC = 2
_NS = 16
NW = _NC * _NS
BPW = B // NW
NBUF = 4

_PERM = [(m // 2) if m % 2 == 0 else 64 + (m // 2) for m in range(BPW)]
_INV_PERM = [0] * BPW
for _m, _k in enumerate(_PERM):
    _INV_PERM[_k] = _m

_mesh = plsc.VectorSubcoreMesh(core_axis_name="c", subcore_axis_name="s")


@functools.partial(
    pl.kernel,
    mesh=_mesh,
    compiler_params=pltpu.CompilerParams(use_tc_tiling_on_sc=False),
    out_type=[
        jax.ShapeDtypeStruct((L * B, D), jnp.float32),
        jax.ShapeDtypeStruct((B, D), jnp.float32),
        jax.ShapeDtypeStruct((B, D), jnp.float32),
    ],
    scratch_types=[
        pltpu.VMEM((L, BPW), jnp.int32),
        pltpu.VMEM((NBUF, BPW, D), jnp.float32),
        pltpu.VMEM((BPW, D), jnp.float32),
        pltpu.VMEM((BPW, D), jnp.float32),
        pltpu.SemaphoreType.DMA((NBUF,)),
        pltpu.SemaphoreType.DMA((NBUF,)),
    ],
)
def _sc_gather_stats(idx_hbm, emb_hbm, raw_hbm, s_hbm, q_hbm,
                     idx_v, rows_v, acc_s, acc_q, gsem, wsem):
    wid = lax.axis_index("s") * _NC + lax.axis_index("c")
    b0 = wid * BPW

    pltpu.sync_copy(idx_hbm.at[wid], idx_v)

    zeros = jnp.zeros((16,), jnp.float32)

    def zero_body(r, _):
        for c in range(D // 16):
            acc_s[r, pl.ds(c * 16, 16)] = zeros
            acc_q[r, pl.ds(c * 16, 16)] = zeros
        return 0
    lax.fori_loop(0, BPW, zero_body, 0)

    def fire_gather(l, j):
        pltpu.async_copy(emb_hbm.at[idx_v.at[l]], rows_v.at[j], gsem.at[j])

    def fire_write(l, j):
        pltpu.async_copy(rows_v.at[j], raw_hbm.at[pl.ds(l * B + b0, BPW)],
                         wsem.at[j])

    def wait_gather(j):
        pltpu.make_async_copy(emb_hbm.at[idx_v.at[0]], rows_v.at[j],
                              gsem.at[j]).wait()

    def wait_write(j):
        pltpu.make_async_copy(rows_v.at[j], raw_hbm.at[pl.ds(b0, BPW)],
                              wsem.at[j]).wait()

    def accumulate(j):
        def r_body(r, _):
            for c in range(D // 16):
                x = rows_v[j, r, pl.ds(c * 16, 16)]
                acc_s[r, pl.ds(c * 16, 16)] += x
                acc_q[r, pl.ds(c * 16, 16)] += x * x
            return 0
        lax.fori_loop(0, BPW, r_body, 0)

    def step(l, j, jprev, first):
        wait_gather(j)
        fire_write(l, j)
        accumulate(j)
        if not first:
            wait_write(jprev)
        fire_gather(l + NBUF - 1, jprev)

    for j in range(NBUF - 1):
        fire_gather(j, j)

    step(0, 0, NBUF - 1, True)
    for l in range(1, NBUF):
        step(l, l % NBUF, (l - 1) % NBUF, False)

    def chunk(c, _):
        base = c * NBUF
        for j in range(NBUF):
            step(base + j, j, (j - 1) % NBUF, False)
        return 0
    lax.fori_loop(1, (L // NBUF) - 1, chunk, 0)

    l = 196
    wait_gather(l % NBUF)
    fire_write(l, l % NBUF)
    accumulate(l % NBUF)
    wait_write((l - 1) % NBUF)
    fire_gather(199, (l - 1) % NBUF)
    for l in (197, 198, 199):
        wait_gather(l % NBUF)
        fire_write(l, l % NBUF)
        accumulate(l % NBUF)

    for j in range(NBUF):
        wait_write(j)

    pltpu.sync_copy(acc_s, s_hbm.at[pl.ds(b0, BPW)])
    pltpu.sync_copy(acc_q, q_hbm.at[pl.ds(b0, BPW)])


def _finalize_body(s_ref, q_ref, a_ref, c_ref):
    s = s_ref[...]
    q = q_ref[...]
    se = jnp.sum(s[:, :, :D], axis=2)
    so = jnp.sum(s[:, :, D:], axis=2)
    qe = jnp.sum(q[:, :, :D], axis=2)
    qo = jnp.sum(q[:, :, D:], axis=2)
    sum_b = jnp.concatenate([se, so], axis=-1)
    sumsq_b = jnp.concatenate([qe, qo], axis=-1)
    n = jnp.float32(N_ELEM)
    mean = sum_b / n
    var = (sumsq_b - sum_b * sum_b / n) / (n - 1.0)
    std = jnp.sqrt(var)
    inv = SCALE / (SCALE * std + EPS)
    a_ref[:, 0, :] = inv
    c_ref[:, 0, :] = -mean * inv


_finalize = pl.pallas_call(
    _finalize_body,
    out_shape=[
        jax.ShapeDtypeStruct((NW, 1, BPW), jnp.float32),
        jax.ShapeDtypeStruct((NW, 1, BPW), jnp.float32),
    ],
)

_LBLK = 8
_WGRP = 8


def _norm_body(x_ref, a_ref, c_ref, o_ref):
    eye = jnp.eye(D, dtype=jnp.float32)
    for k in range(_WGRP):
        av = a_ref[k, 0]
        cv = c_ref[k, 0]
        for l in range(_LBLK):
            xl = x_ref[l, k]
            xt = lax.dot_general(
                xl, eye, dimension_numbers=(((0,), (0,)), ((), ())),
                preferred_element_type=jnp.float32)
            oc = jnp.concatenate([xt[:D, :], xt[D:, :]], axis=1)
            o_ref[l, :, pl.ds(k * 128, 128)] = oc * av + cv


_norm = pl.pallas_call(
    _norm_body,
    grid=(L // _LBLK, NW // _WGRP),
    in_specs=[
        pl.BlockSpec((_LBLK, _WGRP, D, BPW), lambda i, w: (i, w, 0, 0)),
        pl.BlockSpec((_WGRP, 1, BPW), lambda i, w: (w, 0, 0)),
        pl.BlockSpec((_WGRP, 1, BPW), lambda i, w: (w, 0, 0)),
    ],
    out_specs=pl.BlockSpec((_LBLK, D, _WGRP * BPW), lambda i, w: (i, 0, w)),
    out_shape=jax.ShapeDtypeStruct((L, D, B), jnp.float32),
)


def kernel(inp, emb):
    perm = jnp.asarray(_PERM, dtype=jnp.int32)
    idx_w = inp.reshape(NW, BPW, L)[:, perm, :].transpose(0, 2, 1)
    raw, s, q = _sc_gather_stats(idx_w, emb)
    a4, c4 = _finalize(s.reshape(NW, D, BPW), q.reshape(NW, D, BPW))
    y = _norm(raw.reshape(L, NW, D, BPW), a4, c4)
    return jnp.transpose(y, (0, 2, 1))
